# baseline (speedup 1.0000x reference)
import functools

import jax
import jax.numpy as jnp
from jax import lax
from jax.experimental import pallas as pl
from jax.experimental.pallas import tpu as pltpu
from jax.experimental.pallas import tpu_sc as plsc

N_NODES = 10000
N_EDGES = 320000
IN_DIM = 128
H1 = 256
H2 = 64

BN = 1000

NC = 2
NS = 16
NW = NC * NS
CK = 64
NCHUNK = 160
E_PAD = NW * NCHUNK * CK
N_PAD = 10112
RPS = N_PAD // NS

_SC_MESH = plsc.VectorSubcoreMesh(core_axis_name="c", subcore_axis_name="s")


NB_D = 80
CK_D = 128
DN_PAD = 10240
RPS_D = DN_PAD // NS


@functools.partial(
    pl.kernel,
    out_type=jax.ShapeDtypeStruct((NC * DN_PAD,), jnp.float32),
    mesh=_SC_MESH,
    scratch_types=[
        pltpu.VMEM((2, 1, CK_D), jnp.int32),
        pltpu.VMEM((2, 1, CK_D), jnp.float32),
        pltpu.VMEM((640,), jnp.float32),
        pltpu.VMEM_SHARED((DN_PAD,), jnp.float32),
        pltpu.SemaphoreType.DMA,
    ],
)
def _sc_deg(dst_hbm, ew_hbm, out_hbm,
            dst_v, ew_v, zbuf, deg_sh, lsem):
    c = lax.axis_index("c")
    s = lax.axis_index("s")
    wid = s * NC + c

    def zrow(i, carry):
        zbuf[pl.ds(i * 16, 16)] = jnp.zeros((16,), jnp.float32)
        return carry

    lax.fori_loop(0, 640 // 16, zrow, 0)
    pltpu.sync_copy(zbuf, deg_sh.at[pl.ds(s * RPS_D, RPS_D)])
    pltpu.async_copy(dst_hbm.at[wid, 0], dst_v.at[0], lsem)
    pltpu.async_copy(ew_hbm.at[wid, 0], ew_v.at[0], lsem)
    plsc.subcore_barrier()

    def body(b, carry):
        bb = b % 2
        pltpu.make_async_copy(dst_hbm.at[wid, b], dst_v.at[bb], lsem).wait()
        pltpu.make_async_copy(ew_hbm.at[wid, b], ew_v.at[bb], lsem).wait()

        @pl.when(b + 1 < NB_D)
        def _():
            pltpu.async_copy(dst_hbm.at[wid, b + 1], dst_v.at[(b + 1) % 2],
                             lsem)
            pltpu.async_copy(ew_hbm.at[wid, b + 1], ew_v.at[(b + 1) % 2],
                             lsem)

        pltpu.sync_copy(ew_v.at[bb, 0], deg_sh.at[dst_v.at[bb, 0]], add=True)
        return carry

    lax.fori_loop(0, NB_D, body, 0)
    plsc.subcore_barrier()
    pltpu.sync_copy(deg_sh.at[pl.ds(s * RPS_D, RPS_D)],
                    out_hbm.at[pl.ds(c * DN_PAD + s * RPS_D, RPS_D)])


@functools.partial(
    pl.kernel,
    out_type=jax.ShapeDtypeStruct((NC, N_PAD, IN_DIM), jnp.float32),
    mesh=_SC_MESH,
    scratch_types=[
        pltpu.VMEM((4, 1, CK), jnp.int32),
        pltpu.VMEM((4, 1, CK), jnp.int32),
        pltpu.VMEM((3, CK, 16), jnp.float32),
        pltpu.VMEM((3, CK, IN_DIM), jnp.float32),
        pltpu.VMEM_SHARED((N_PAD, IN_DIM), jnp.float32),
        pltpu.SemaphoreType.DMA,
        pltpu.SemaphoreType.DMA,
        pltpu.SemaphoreType.DMA,
    ],
)
def _sc_pass(y_hbm, src_hbm, dstl_hbm, w_hbm, out_hbm,
             es_v, ed_v, w_v, rows_v, acc_sh, esem, gsem, ssem):
    c = lax.axis_index("c")
    s = lax.axis_index("s")
    wid = s * NC + c
    sr_hbm = src_hbm.at[wid]
    dl_hbm = dstl_hbm.at[wid]
    wf_hbm = w_hbm.at[wid]

    def zrow(i, carry):
        for f in range(IN_DIM // 16):
            rows_v[0, i, pl.ds(f * 16, 16)] = jnp.zeros((16,), jnp.float32)
        return carry

    lax.fori_loop(0, CK, zrow, 0)

    def zcp(t, carry):
        pltpu.sync_copy(rows_v.at[0], acc_sh.at[pl.ds(s * RPS + t * CK, CK)])
        return carry

    lax.fori_loop(0, RPS // CK, zcp, 0)

    def zcp_tail(t, carry):
        pltpu.sync_copy(
            rows_v.at[0, pl.ds(0, 8)],
            acc_sh.at[pl.ds(s * RPS + (RPS // CK) * CK + t * 8, 8)])
        return carry

    lax.fori_loop(0, (RPS % CK) // 8, zcp_tail, 0)
    plsc.subcore_barrier()

    pltpu.sync_copy(sr_hbm.at[0], es_v.at[0])
    pltpu.sync_copy(dl_hbm.at[0], ed_v.at[0])
    pltpu.sync_copy(wf_hbm.at[0], w_v.at[0])
    pltpu.async_copy(y_hbm.at[es_v.at[0, 0]], rows_v.at[0], gsem)
    pltpu.async_copy(sr_hbm.at[1], es_v.at[1], esem)
    pltpu.async_copy(dl_hbm.at[1], ed_v.at[1], esem)
    pltpu.async_copy(wf_hbm.at[1], w_v.at[1], esem)

    def body(k, carry):
        eb = k % 4
        wb = k % 3
        rb = k % 3

        @pl.when(k + 1 < NCHUNK)
        def _():
            @pl.when(k >= 2)
            def _():
                kk = k - 2
                pltpu.make_async_copy(
                    rows_v.at[kk % 3], acc_sh.at[ed_v.at[kk % 4, 0]],
                    ssem).wait()

            pltpu.make_async_copy(sr_hbm.at[k + 1], es_v.at[(k + 1) % 4],
                                  esem).wait()
            pltpu.make_async_copy(dl_hbm.at[k + 1], ed_v.at[(k + 1) % 4],
                                  esem).wait()
            pltpu.make_async_copy(wf_hbm.at[k + 1], w_v.at[(k + 1) % 3],
                                  esem).wait()
            pltpu.async_copy(y_hbm.at[es_v.at[(k + 1) % 4, 0]],
                             rows_v.at[(k + 1) % 3], gsem)

        @pl.when(k + 2 < NCHUNK)
        def _():
            pltpu.async_copy(sr_hbm.at[k + 2], es_v.at[(k + 2) % 4], esem)
            pltpu.async_copy(dl_hbm.at[k + 2], ed_v.at[(k + 2) % 4], esem)
            pltpu.async_copy(wf_hbm.at[k + 2], w_v.at[(k + 2) % 3], esem)

        pltpu.make_async_copy(y_hbm.at[es_v.at[eb, 0]], rows_v.at[rb],
                              gsem).wait()

        def scale(r, carry2):
            wrow = w_v[wb, r]
            for f in range(IN_DIM // 16):
                sl = pl.ds(f * 16, 16)
                rows_v[rb, r, sl] = rows_v[rb, r, sl] * wrow
            return carry2

        lax.fori_loop(0, CK, scale, 0)
        pltpu.async_copy(rows_v.at[rb], acc_sh.at[ed_v.at[eb, 0]], ssem,
                         add=True)
        return carry

    lax.fori_loop(0, NCHUNK, body, 0)

    def drain(t, carry):
        pltpu.make_async_copy(rows_v.at[t % 3], acc_sh.at[ed_v.at[t % 4, 0]],
                              ssem).wait()
        return carry

    lax.fori_loop(NCHUNK - 3, NCHUNK, drain, 0)
    plsc.subcore_barrier()
    pltpu.sync_copy(acc_sh.at[pl.ds(s * RPS, RPS)],
                    out_hbm.at[c, pl.ds(s * RPS, RPS)])


def _t1_body(d0_ref, d1_ref, x_ref, dinv_ref, y1_ref):
    deg = d0_ref[...] + d1_ref[...] + 1.0
    dinv = jax.lax.rsqrt(deg)
    dinv_ref[...] = dinv
    y1_ref[...] = x_ref[...] * dinv


def _t1(d0, d1, x):
    return pl.pallas_call(
        _t1_body,
        grid=(N_NODES // BN,),
        in_specs=[
            pl.BlockSpec((BN, 1), lambda i: (i, 0)),
            pl.BlockSpec((BN, 1), lambda i: (i, 0)),
            pl.BlockSpec((BN, IN_DIM), lambda i: (i, 0)),
        ],
        out_specs=[
            pl.BlockSpec((BN, 1), lambda i: (i, 0)),
            pl.BlockSpec((BN, IN_DIM), lambda i: (i, 0)),
        ],
        out_shape=[
            jax.ShapeDtypeStruct((N_NODES, 1), jnp.float32),
            jax.ShapeDtypeStruct((N_NODES, IN_DIM), jnp.float32),
        ],
    )(d0, d1, x)


def _t2_body(a0_ref, a1_ref, y1_ref, dinv_ref, w1_ref, b1_ref, w23_ref,
             y2_ref):
    dinv = dinv_ref[...]
    ax = dinv * (a0_ref[...] + a1_ref[...] + y1_ref[...])
    h = jax.nn.relu(
        jnp.dot(ax, w1_ref[...], preferred_element_type=jnp.float32)
        + b1_ref[...]
    )
    hc = jnp.dot(h, w23_ref[...], preferred_element_type=jnp.float32)
    y2_ref[...] = hc * dinv


def _t2(a0, a1, y1, dinv, W1, b1, W23):
    return pl.pallas_call(
        _t2_body,
        grid=(N_NODES // BN,),
        in_specs=[
            pl.BlockSpec((BN, IN_DIM), lambda i: (i, 0)),
            pl.BlockSpec((BN, IN_DIM), lambda i: (i, 0)),
            pl.BlockSpec((BN, IN_DIM), lambda i: (i, 0)),
            pl.BlockSpec((BN, 1), lambda i: (i, 0)),
            pl.BlockSpec((IN_DIM, H1), lambda i: (0, 0)),
            pl.BlockSpec((1, H1), lambda i: (0, 0)),
            pl.BlockSpec((H1, 2 * H2), lambda i: (0, 0)),
        ],
        out_specs=pl.BlockSpec((BN, 2 * H2), lambda i: (i, 0)),
        out_shape=jax.ShapeDtypeStruct((N_NODES, 2 * H2), jnp.float32),
    )(a0, a1, y1, dinv, W1, b1, W23)


def _t3_body(c0_ref, c1_ref, y2_ref, dinv_ref, b23_ref, noise_ref,
             z_ref, mean_ref, ls_ref):
    out2 = (dinv_ref[...] * (c0_ref[...] + c1_ref[...] + y2_ref[...])
            + b23_ref[...])
    mean = out2[:, :H2]
    log_std = out2[:, H2:]
    mean_ref[...] = mean
    ls_ref[...] = log_std
    z_ref[...] = mean + noise_ref[...] * jnp.exp(log_std)


def _t3(c0, c1, y2, dinv, b23, noise):
    return pl.pallas_call(
        _t3_body,
        grid=(N_NODES // BN,),
        in_specs=[
            pl.BlockSpec((BN, 2 * H2), lambda i: (i, 0)),
            pl.BlockSpec((BN, 2 * H2), lambda i: (i, 0)),
            pl.BlockSpec((BN, 2 * H2), lambda i: (i, 0)),
            pl.BlockSpec((BN, 1), lambda i: (i, 0)),
            pl.BlockSpec((1, 2 * H2), lambda i: (0, 0)),
            pl.BlockSpec((BN, H2), lambda i: (i, 0)),
        ],
        out_specs=[
            pl.BlockSpec((BN, H2), lambda i: (i, 0)),
            pl.BlockSpec((BN, H2), lambda i: (i, 0)),
            pl.BlockSpec((BN, H2), lambda i: (i, 0)),
        ],
        out_shape=[
            jax.ShapeDtypeStruct((N_NODES, H2), jnp.float32),
            jax.ShapeDtypeStruct((N_NODES, H2), jnp.float32),
            jax.ShapeDtypeStruct((N_NODES, H2), jnp.float32),
        ],
    )(c0, c1, y2, dinv, b23, noise)


def kernel(x, edge_index, edge_weight, noise, W1, b1, W2, b2, W3, b3):
    n_pad = E_PAD - N_EDGES
    pad_idx = jnp.arange(n_pad, dtype=jnp.int32) % N_NODES
    src = jnp.concatenate([edge_index[0].astype(jnp.int32), pad_idx])
    dst = jnp.concatenate([edge_index[1].astype(jnp.int32), pad_idx])
    ew = jnp.concatenate([edge_weight, jnp.zeros((n_pad,), jnp.float32)])
    src3 = src.reshape(NW, NCHUNK, 1, CK)
    dst3 = dst.reshape(NW, NCHUNK, 1, CK)
    w3 = ew.reshape(NW, NCHUNK, CK)
    w3r = jnp.broadcast_to(w3[..., None], (NW, NCHUNK, CK, 16))

    deg = _sc_deg(dst.reshape(NW, NB_D, 1, CK_D),
                  ew.reshape(NW, NB_D, 1, CK_D)).reshape(NC, DN_PAD)

    dinv, y1 = _t1(deg[0].reshape(DN_PAD, 1), deg[1].reshape(DN_PAD, 1), x)

    a = _sc_pass(y1, src3, dst3, w3r)
    a0, a1 = a[0], a[1]

    W23 = jnp.concatenate([W2, W3], axis=1)
    y2 = _t2(a0, a1, y1, dinv, W1, b1.reshape(1, H1), W23)

    cc = _sc_pass(y2, src3, dst3, w3r)
    c0, c1 = cc[0], cc[1]

    b23 = jnp.concatenate([b2, b3]).reshape(1, 2 * H2)
    z, mean, log_std = _t3(c0, c1, y2, dinv, b23, noise)
    return (z, mean, log_std)


# speedup vs baseline: 1.7596x; 1.7596x over previous
import functools

import jax
import jax.numpy as jnp
from jax import lax
from jax.experimental import pallas as pl
from jax.experimental.pallas import tpu as pltpu
from jax.experimental.pallas import tpu_sc as plsc

N_NODES = 10000
N_EDGES = 320000
IN_DIM = 128
H1 = 256
H2 = 64

BN = 1000

NC = 2
NS = 16
NW = NC * NS
CK = 64
NCHUNK = 160
E_PAD = NW * NCHUNK * CK
N_PAD = 10112
RPS = N_PAD // NS

_SC_MESH = plsc.VectorSubcoreMesh(core_axis_name="c", subcore_axis_name="s")


NB_D = 80
CK_D = 128
DN_PAD = 10240
RPS_D = DN_PAD // NS


@functools.partial(
    pl.kernel,
    out_type=jax.ShapeDtypeStruct((NC * DN_PAD,), jnp.float32),
    mesh=_SC_MESH,
    scratch_types=[
        pltpu.VMEM((2, 1, CK_D), jnp.int32),
        pltpu.VMEM((2, 1, CK_D), jnp.float32),
        pltpu.VMEM((640,), jnp.float32),
        pltpu.VMEM_SHARED((DN_PAD,), jnp.float32),
        pltpu.SemaphoreType.DMA,
    ],
)
def _sc_deg(dst_hbm, ew_hbm, out_hbm,
            dst_v, ew_v, zbuf, deg_sh, lsem):
    c = lax.axis_index("c")
    s = lax.axis_index("s")
    wid = s * NC + c

    def zrow(i, carry):
        zbuf[pl.ds(i * 16, 16)] = jnp.zeros((16,), jnp.float32)
        return carry

    lax.fori_loop(0, 640 // 16, zrow, 0)
    pltpu.sync_copy(zbuf, deg_sh.at[pl.ds(s * RPS_D, RPS_D)])
    pltpu.async_copy(dst_hbm.at[wid, 0], dst_v.at[0], lsem)
    pltpu.async_copy(ew_hbm.at[wid, 0], ew_v.at[0], lsem)
    plsc.subcore_barrier()

    def body(b, carry):
        bb = b % 2
        pltpu.make_async_copy(dst_hbm.at[wid, b], dst_v.at[bb], lsem).wait()
        pltpu.make_async_copy(ew_hbm.at[wid, b], ew_v.at[bb], lsem).wait()

        @pl.when(b + 1 < NB_D)
        def _():
            pltpu.async_copy(dst_hbm.at[wid, b + 1], dst_v.at[(b + 1) % 2],
                             lsem)
            pltpu.async_copy(ew_hbm.at[wid, b + 1], ew_v.at[(b + 1) % 2],
                             lsem)

        pltpu.sync_copy(ew_v.at[bb, 0], deg_sh.at[dst_v.at[bb, 0]], add=True)
        return carry

    lax.fori_loop(0, NB_D, body, 0)
    plsc.subcore_barrier()
    pltpu.sync_copy(deg_sh.at[pl.ds(s * RPS_D, RPS_D)],
                    out_hbm.at[pl.ds(c * DN_PAD + s * RPS_D, RPS_D)])


@functools.partial(
    pl.kernel,
    out_type=jax.ShapeDtypeStruct((NC, N_PAD, IN_DIM), jnp.float32),
    mesh=_SC_MESH,
    scratch_types=[
        pltpu.VMEM((4, 1, CK), jnp.int32),
        pltpu.VMEM((4, 1, CK), jnp.int32),
        pltpu.VMEM((3, CK, 16), jnp.float32),
        pltpu.VMEM((3, CK, IN_DIM), jnp.float32),
        pltpu.VMEM_SHARED((N_PAD, IN_DIM), jnp.float32),
        pltpu.SemaphoreType.DMA,
        pltpu.SemaphoreType.DMA,
        pltpu.SemaphoreType.DMA,
    ],
)
def _sc_pass(y_hbm, src_hbm, dstl_hbm, w_hbm, out_hbm,
             es_v, ed_v, w_v, rows_v, acc_sh, esem, gsem, ssem):
    c = lax.axis_index("c")
    s = lax.axis_index("s")
    wid = s * NC + c
    sr_hbm = src_hbm.at[wid]
    dl_hbm = dstl_hbm.at[wid]
    wf_hbm = w_hbm.at[wid]

    def zrow(i, carry):
        for f in range(IN_DIM // 16):
            rows_v[0, i, pl.ds(f * 16, 16)] = jnp.zeros((16,), jnp.float32)
        return carry

    lax.fori_loop(0, CK, zrow, 0)

    def zcp(t, carry):
        pltpu.sync_copy(rows_v.at[0], acc_sh.at[pl.ds(s * RPS + t * CK, CK)])
        return carry

    lax.fori_loop(0, RPS // CK, zcp, 0)

    def zcp_tail(t, carry):
        pltpu.sync_copy(
            rows_v.at[0, pl.ds(0, 8)],
            acc_sh.at[pl.ds(s * RPS + (RPS // CK) * CK + t * 8, 8)])
        return carry

    lax.fori_loop(0, (RPS % CK) // 8, zcp_tail, 0)
    plsc.subcore_barrier()

    pltpu.sync_copy(sr_hbm.at[0], es_v.at[0])
    pltpu.sync_copy(dl_hbm.at[0], ed_v.at[0])
    pltpu.sync_copy(wf_hbm.at[0], w_v.at[0])
    pltpu.async_copy(y_hbm.at[es_v.at[0, 0]], rows_v.at[0], gsem)
    pltpu.async_copy(sr_hbm.at[1], es_v.at[1], esem)
    pltpu.async_copy(dl_hbm.at[1], ed_v.at[1], esem)
    pltpu.async_copy(wf_hbm.at[1], w_v.at[1], esem)

    def body(k, carry):
        eb = k % 4
        wb = k % 3
        rb = k % 3

        @pl.when(k + 1 < NCHUNK)
        def _():
            @pl.when(k >= 2)
            def _():
                kk = k - 2
                pltpu.make_async_copy(
                    rows_v.at[kk % 3], acc_sh.at[ed_v.at[kk % 4, 0]],
                    ssem).wait()

            pltpu.make_async_copy(sr_hbm.at[k + 1], es_v.at[(k + 1) % 4],
                                  esem).wait()
            pltpu.make_async_copy(dl_hbm.at[k + 1], ed_v.at[(k + 1) % 4],
                                  esem).wait()
            pltpu.make_async_copy(wf_hbm.at[k + 1], w_v.at[(k + 1) % 3],
                                  esem).wait()
            pltpu.async_copy(y_hbm.at[es_v.at[(k + 1) % 4, 0]],
                             rows_v.at[(k + 1) % 3], gsem)

        @pl.when(k + 2 < NCHUNK)
        def _():
            pltpu.async_copy(sr_hbm.at[k + 2], es_v.at[(k + 2) % 4], esem)
            pltpu.async_copy(dl_hbm.at[k + 2], ed_v.at[(k + 2) % 4], esem)
            pltpu.async_copy(wf_hbm.at[k + 2], w_v.at[(k + 2) % 3], esem)

        pltpu.make_async_copy(y_hbm.at[es_v.at[eb, 0]], rows_v.at[rb],
                              gsem).wait()

        @plsc.parallel_loop(0, CK, unroll=8)
        def _scale(r):
            wrow = w_v[wb, r]
            for f in range(IN_DIM // 16):
                sl = pl.ds(f * 16, 16)
                rows_v[rb, r, sl] = rows_v[rb, r, sl] * wrow
        pltpu.async_copy(rows_v.at[rb], acc_sh.at[ed_v.at[eb, 0]], ssem,
                         add=True)
        return carry

    lax.fori_loop(0, NCHUNK, body, 0)

    def drain(t, carry):
        pltpu.make_async_copy(rows_v.at[t % 3], acc_sh.at[ed_v.at[t % 4, 0]],
                              ssem).wait()
        return carry

    lax.fori_loop(NCHUNK - 3, NCHUNK, drain, 0)
    plsc.subcore_barrier()
    pltpu.sync_copy(acc_sh.at[pl.ds(s * RPS, RPS)],
                    out_hbm.at[c, pl.ds(s * RPS, RPS)])


def _t1_body(d0_ref, d1_ref, x_ref, dinv_ref, y1_ref):
    deg = d0_ref[...] + d1_ref[...] + 1.0
    dinv = jax.lax.rsqrt(deg)
    dinv_ref[...] = dinv
    y1_ref[...] = x_ref[...] * dinv


def _t1(d0, d1, x):
    return pl.pallas_call(
        _t1_body,
        grid=(N_NODES // BN,),
        in_specs=[
            pl.BlockSpec((BN, 1), lambda i: (i, 0)),
            pl.BlockSpec((BN, 1), lambda i: (i, 0)),
            pl.BlockSpec((BN, IN_DIM), lambda i: (i, 0)),
        ],
        out_specs=[
            pl.BlockSpec((BN, 1), lambda i: (i, 0)),
            pl.BlockSpec((BN, IN_DIM), lambda i: (i, 0)),
        ],
        out_shape=[
            jax.ShapeDtypeStruct((N_NODES, 1), jnp.float32),
            jax.ShapeDtypeStruct((N_NODES, IN_DIM), jnp.float32),
        ],
    )(d0, d1, x)


def _t2_body(a0_ref, a1_ref, y1_ref, dinv_ref, w1_ref, b1_ref, w23_ref,
             y2_ref):
    dinv = dinv_ref[...]
    ax = dinv * (a0_ref[...] + a1_ref[...] + y1_ref[...])
    h = jax.nn.relu(
        jnp.dot(ax, w1_ref[...], preferred_element_type=jnp.float32)
        + b1_ref[...]
    )
    hc = jnp.dot(h, w23_ref[...], preferred_element_type=jnp.float32)
    y2_ref[...] = hc * dinv


def _t2(a0, a1, y1, dinv, W1, b1, W23):
    return pl.pallas_call(
        _t2_body,
        grid=(N_NODES // BN,),
        in_specs=[
            pl.BlockSpec((BN, IN_DIM), lambda i: (i, 0)),
            pl.BlockSpec((BN, IN_DIM), lambda i: (i, 0)),
            pl.BlockSpec((BN, IN_DIM), lambda i: (i, 0)),
            pl.BlockSpec((BN, 1), lambda i: (i, 0)),
            pl.BlockSpec((IN_DIM, H1), lambda i: (0, 0)),
            pl.BlockSpec((1, H1), lambda i: (0, 0)),
            pl.BlockSpec((H1, 2 * H2), lambda i: (0, 0)),
        ],
        out_specs=pl.BlockSpec((BN, 2 * H2), lambda i: (i, 0)),
        out_shape=jax.ShapeDtypeStruct((N_NODES, 2 * H2), jnp.float32),
    )(a0, a1, y1, dinv, W1, b1, W23)


def _t3_body(c0_ref, c1_ref, y2_ref, dinv_ref, b23_ref, noise_ref,
             z_ref, mean_ref, ls_ref):
    out2 = (dinv_ref[...] * (c0_ref[...] + c1_ref[...] + y2_ref[...])
            + b23_ref[...])
    mean = out2[:, :H2]
    log_std = out2[:, H2:]
    mean_ref[...] = mean
    ls_ref[...] = log_std
    z_ref[...] = mean + noise_ref[...] * jnp.exp(log_std)


def _t3(c0, c1, y2, dinv, b23, noise):
    return pl.pallas_call(
        _t3_body,
        grid=(N_NODES // BN,),
        in_specs=[
            pl.BlockSpec((BN, 2 * H2), lambda i: (i, 0)),
            pl.BlockSpec((BN, 2 * H2), lambda i: (i, 0)),
            pl.BlockSpec((BN, 2 * H2), lambda i: (i, 0)),
            pl.BlockSpec((BN, 1), lambda i: (i, 0)),
            pl.BlockSpec((1, 2 * H2), lambda i: (0, 0)),
            pl.BlockSpec((BN, H2), lambda i: (i, 0)),
        ],
        out_specs=[
            pl.BlockSpec((BN, H2), lambda i: (i, 0)),
            pl.BlockSpec((BN, H2), lambda i: (i, 0)),
            pl.BlockSpec((BN, H2), lambda i: (i, 0)),
        ],
        out_shape=[
            jax.ShapeDtypeStruct((N_NODES, H2), jnp.float32),
            jax.ShapeDtypeStruct((N_NODES, H2), jnp.float32),
            jax.ShapeDtypeStruct((N_NODES, H2), jnp.float32),
        ],
    )(c0, c1, y2, dinv, b23, noise)


def kernel(x, edge_index, edge_weight, noise, W1, b1, W2, b2, W3, b3):
    n_pad = E_PAD - N_EDGES
    pad_idx = jnp.arange(n_pad, dtype=jnp.int32) % N_NODES
    src = jnp.concatenate([edge_index[0].astype(jnp.int32), pad_idx])
    dst = jnp.concatenate([edge_index[1].astype(jnp.int32), pad_idx])
    ew = jnp.concatenate([edge_weight, jnp.zeros((n_pad,), jnp.float32)])
    src3 = src.reshape(NW, NCHUNK, 1, CK)
    dst3 = dst.reshape(NW, NCHUNK, 1, CK)
    w3 = ew.reshape(NW, NCHUNK, CK)
    w3r = jnp.broadcast_to(w3[..., None], (NW, NCHUNK, CK, 16))

    deg = _sc_deg(dst.reshape(NW, NB_D, 1, CK_D),
                  ew.reshape(NW, NB_D, 1, CK_D)).reshape(NC, DN_PAD)

    dinv, y1 = _t1(deg[0].reshape(DN_PAD, 1), deg[1].reshape(DN_PAD, 1), x)

    a = _sc_pass(y1, src3, dst3, w3r)
    a0, a1 = a[0], a[1]

    W23 = jnp.concatenate([W2, W3], axis=1)
    y2 = _t2(a0, a1, y1, dinv, W1, b1.reshape(1, H1), W23)

    cc = _sc_pass(y2, src3, dst3, w3r)
    c0, c1 = cc[0], cc[1]

    b23 = jnp.concatenate([b2, b3]).reshape(1, 2 * H2)
    z, mean, log_std = _t3(c0, c1, y2, dinv, b23, noise)
    return (z, mean, log_std)


# speedup vs baseline: 1.7603x; 1.0004x over previous
import functools

import jax
import jax.numpy as jnp
from jax import lax
from jax.experimental import pallas as pl
from jax.experimental.pallas import tpu as pltpu
from jax.experimental.pallas import tpu_sc as plsc

N_NODES = 10000
N_EDGES = 320000
IN_DIM = 128
H1 = 256
H2 = 64

BN = 1000

NC = 2
NS = 16
NW = NC * NS
CK = 64
NCHUNK = 160
E_PAD = NW * NCHUNK * CK
N_PAD = 10112
RPS = N_PAD // NS

_SC_MESH = plsc.VectorSubcoreMesh(core_axis_name="c", subcore_axis_name="s")


NB_D = 80
CK_D = 128
DN_PAD = 10240
RPS_D = DN_PAD // NS


@functools.partial(
    pl.kernel,
    out_type=jax.ShapeDtypeStruct((NC * DN_PAD,), jnp.float32),
    mesh=_SC_MESH,
    scratch_types=[
        pltpu.VMEM((2, 1, CK_D), jnp.int32),
        pltpu.VMEM((2, 1, CK_D), jnp.float32),
        pltpu.VMEM((640,), jnp.float32),
        pltpu.VMEM_SHARED((DN_PAD,), jnp.float32),
        pltpu.SemaphoreType.DMA,
    ],
)
def _sc_deg(dst_hbm, ew_hbm, out_hbm,
            dst_v, ew_v, zbuf, deg_sh, lsem):
    c = lax.axis_index("c")
    s = lax.axis_index("s")
    wid = s * NC + c

    def zrow(i, carry):
        zbuf[pl.ds(i * 16, 16)] = jnp.zeros((16,), jnp.float32)
        return carry

    lax.fori_loop(0, 640 // 16, zrow, 0)
    pltpu.sync_copy(zbuf, deg_sh.at[pl.ds(s * RPS_D, RPS_D)])
    pltpu.async_copy(dst_hbm.at[wid, 0], dst_v.at[0], lsem)
    pltpu.async_copy(ew_hbm.at[wid, 0], ew_v.at[0], lsem)
    plsc.subcore_barrier()

    def body(b, carry):
        bb = b % 2
        pltpu.make_async_copy(dst_hbm.at[wid, b], dst_v.at[bb], lsem).wait()
        pltpu.make_async_copy(ew_hbm.at[wid, b], ew_v.at[bb], lsem).wait()

        @pl.when(b + 1 < NB_D)
        def _():
            pltpu.async_copy(dst_hbm.at[wid, b + 1], dst_v.at[(b + 1) % 2],
                             lsem)
            pltpu.async_copy(ew_hbm.at[wid, b + 1], ew_v.at[(b + 1) % 2],
                             lsem)

        pltpu.sync_copy(ew_v.at[bb, 0], deg_sh.at[dst_v.at[bb, 0]], add=True)
        return carry

    lax.fori_loop(0, NB_D, body, 0)
    plsc.subcore_barrier()
    pltpu.sync_copy(deg_sh.at[pl.ds(s * RPS_D, RPS_D)],
                    out_hbm.at[pl.ds(c * DN_PAD + s * RPS_D, RPS_D)])


@functools.partial(
    pl.kernel,
    out_type=jax.ShapeDtypeStruct((NC, N_PAD, IN_DIM), jnp.float32),
    mesh=_SC_MESH,
    scratch_types=[
        pltpu.VMEM((4, 1, CK), jnp.int32),
        pltpu.VMEM((4, 1, CK), jnp.int32),
        pltpu.VMEM((3, CK, 16), jnp.float32),
        pltpu.VMEM((3, CK, IN_DIM), jnp.float32),
        pltpu.VMEM_SHARED((N_PAD, IN_DIM), jnp.float32),
        pltpu.SemaphoreType.DMA,
        pltpu.SemaphoreType.DMA,
        pltpu.SemaphoreType.DMA,
    ],
)
def _sc_pass(y_hbm, src_hbm, dstl_hbm, w_hbm, out_hbm,
             es_v, ed_v, w_v, rows_v, acc_sh, esem, gsem, ssem):
    c = lax.axis_index("c")
    s = lax.axis_index("s")
    wid = s * NC + c
    sr_hbm = src_hbm.at[wid]
    dl_hbm = dstl_hbm.at[wid]
    wf_hbm = w_hbm.at[wid]

    def zrow(i, carry):
        for f in range(IN_DIM // 16):
            rows_v[0, i, pl.ds(f * 16, 16)] = jnp.zeros((16,), jnp.float32)
        return carry

    lax.fori_loop(0, CK, zrow, 0)

    def zcp(t, carry):
        pltpu.sync_copy(rows_v.at[0], acc_sh.at[pl.ds(s * RPS + t * CK, CK)])
        return carry

    lax.fori_loop(0, RPS // CK, zcp, 0)

    def zcp_tail(t, carry):
        pltpu.sync_copy(
            rows_v.at[0, pl.ds(0, 8)],
            acc_sh.at[pl.ds(s * RPS + (RPS // CK) * CK + t * 8, 8)])
        return carry

    lax.fori_loop(0, (RPS % CK) // 8, zcp_tail, 0)
    plsc.subcore_barrier()

    pltpu.sync_copy(sr_hbm.at[0], es_v.at[0])
    pltpu.sync_copy(dl_hbm.at[0], ed_v.at[0])
    pltpu.sync_copy(wf_hbm.at[0], w_v.at[0])
    pltpu.async_copy(y_hbm.at[es_v.at[0, 0]], rows_v.at[0], gsem)
    pltpu.async_copy(sr_hbm.at[1], es_v.at[1], esem)
    pltpu.async_copy(dl_hbm.at[1], ed_v.at[1], esem)
    pltpu.async_copy(wf_hbm.at[1], w_v.at[1], esem)

    def body(k, carry):
        eb = k % 4
        wb = k % 3
        rb = k % 3

        @pl.when(k + 1 < NCHUNK)
        def _():
            @pl.when(k >= 2)
            def _():
                kk = k - 2
                pltpu.make_async_copy(
                    rows_v.at[kk % 3], acc_sh.at[ed_v.at[kk % 4, 0]],
                    ssem).wait()

            pltpu.make_async_copy(sr_hbm.at[k + 1], es_v.at[(k + 1) % 4],
                                  esem).wait()
            pltpu.make_async_copy(dl_hbm.at[k + 1], ed_v.at[(k + 1) % 4],
                                  esem).wait()
            pltpu.make_async_copy(wf_hbm.at[k + 1], w_v.at[(k + 1) % 3],
                                  esem).wait()
            pltpu.async_copy(y_hbm.at[es_v.at[(k + 1) % 4, 0]],
                             rows_v.at[(k + 1) % 3], gsem)

        @pl.when(k + 2 < NCHUNK)
        def _():
            pltpu.async_copy(sr_hbm.at[k + 2], es_v.at[(k + 2) % 4], esem)
            pltpu.async_copy(dl_hbm.at[k + 2], ed_v.at[(k + 2) % 4], esem)
            pltpu.async_copy(wf_hbm.at[k + 2], w_v.at[(k + 2) % 3], esem)

        pltpu.make_async_copy(y_hbm.at[es_v.at[eb, 0]], rows_v.at[rb],
                              gsem).wait()

        @plsc.parallel_loop(0, CK, unroll=16)
        def _scale(r):
            wrow = w_v[wb, r]
            for f in range(IN_DIM // 16):
                sl = pl.ds(f * 16, 16)
                rows_v[rb, r, sl] = rows_v[rb, r, sl] * wrow
        pltpu.async_copy(rows_v.at[rb], acc_sh.at[ed_v.at[eb, 0]], ssem,
                         add=True)
        return carry

    lax.fori_loop(0, NCHUNK, body, 0)

    def drain(t, carry):
        pltpu.make_async_copy(rows_v.at[t % 3], acc_sh.at[ed_v.at[t % 4, 0]],
                              ssem).wait()
        return carry

    lax.fori_loop(NCHUNK - 3, NCHUNK, drain, 0)
    plsc.subcore_barrier()
    pltpu.sync_copy(acc_sh.at[pl.ds(s * RPS, RPS)],
                    out_hbm.at[c, pl.ds(s * RPS, RPS)])


def _t1_body(d0_ref, d1_ref, x_ref, dinv_ref, y1_ref):
    deg = d0_ref[...] + d1_ref[...] + 1.0
    dinv = jax.lax.rsqrt(deg)
    dinv_ref[...] = dinv
    y1_ref[...] = x_ref[...] * dinv


def _t1(d0, d1, x):
    return pl.pallas_call(
        _t1_body,
        grid=(N_NODES // BN,),
        in_specs=[
            pl.BlockSpec((BN, 1), lambda i: (i, 0)),
            pl.BlockSpec((BN, 1), lambda i: (i, 0)),
            pl.BlockSpec((BN, IN_DIM), lambda i: (i, 0)),
        ],
        out_specs=[
            pl.BlockSpec((BN, 1), lambda i: (i, 0)),
            pl.BlockSpec((BN, IN_DIM), lambda i: (i, 0)),
        ],
        out_shape=[
            jax.ShapeDtypeStruct((N_NODES, 1), jnp.float32),
            jax.ShapeDtypeStruct((N_NODES, IN_DIM), jnp.float32),
        ],
    )(d0, d1, x)


def _t2_body(a0_ref, a1_ref, y1_ref, dinv_ref, w1_ref, b1_ref, w23_ref,
             y2_ref):
    dinv = dinv_ref[...]
    ax = dinv * (a0_ref[...] + a1_ref[...] + y1_ref[...])
    h = jax.nn.relu(
        jnp.dot(ax, w1_ref[...], preferred_element_type=jnp.float32)
        + b1_ref[...]
    )
    hc = jnp.dot(h, w23_ref[...], preferred_element_type=jnp.float32)
    y2_ref[...] = hc * dinv


def _t2(a0, a1, y1, dinv, W1, b1, W23):
    return pl.pallas_call(
        _t2_body,
        grid=(N_NODES // BN,),
        in_specs=[
            pl.BlockSpec((BN, IN_DIM), lambda i: (i, 0)),
            pl.BlockSpec((BN, IN_DIM), lambda i: (i, 0)),
            pl.BlockSpec((BN, IN_DIM), lambda i: (i, 0)),
            pl.BlockSpec((BN, 1), lambda i: (i, 0)),
            pl.BlockSpec((IN_DIM, H1), lambda i: (0, 0)),
            pl.BlockSpec((1, H1), lambda i: (0, 0)),
            pl.BlockSpec((H1, 2 * H2), lambda i: (0, 0)),
        ],
        out_specs=pl.BlockSpec((BN, 2 * H2), lambda i: (i, 0)),
        out_shape=jax.ShapeDtypeStruct((N_NODES, 2 * H2), jnp.float32),
    )(a0, a1, y1, dinv, W1, b1, W23)


def _t3_body(c0_ref, c1_ref, y2_ref, dinv_ref, b23_ref, noise_ref,
             z_ref, mean_ref, ls_ref):
    out2 = (dinv_ref[...] * (c0_ref[...] + c1_ref[...] + y2_ref[...])
            + b23_ref[...])
    mean = out2[:, :H2]
    log_std = out2[:, H2:]
    mean_ref[...] = mean
    ls_ref[...] = log_std
    z_ref[...] = mean + noise_ref[...] * jnp.exp(log_std)


def _t3(c0, c1, y2, dinv, b23, noise):
    return pl.pallas_call(
        _t3_body,
        grid=(N_NODES // BN,),
        in_specs=[
            pl.BlockSpec((BN, 2 * H2), lambda i: (i, 0)),
            pl.BlockSpec((BN, 2 * H2), lambda i: (i, 0)),
            pl.BlockSpec((BN, 2 * H2), lambda i: (i, 0)),
            pl.BlockSpec((BN, 1), lambda i: (i, 0)),
            pl.BlockSpec((1, 2 * H2), lambda i: (0, 0)),
            pl.BlockSpec((BN, H2), lambda i: (i, 0)),
        ],
        out_specs=[
            pl.BlockSpec((BN, H2), lambda i: (i, 0)),
            pl.BlockSpec((BN, H2), lambda i: (i, 0)),
            pl.BlockSpec((BN, H2), lambda i: (i, 0)),
        ],
        out_shape=[
            jax.ShapeDtypeStruct((N_NODES, H2), jnp.float32),
            jax.ShapeDtypeStruct((N_NODES, H2), jnp.float32),
            jax.ShapeDtypeStruct((N_NODES, H2), jnp.float32),
        ],
    )(c0, c1, y2, dinv, b23, noise)


def kernel(x, edge_index, edge_weight, noise, W1, b1, W2, b2, W3, b3):
    n_pad = E_PAD - N_EDGES
    pad_idx = jnp.arange(n_pad, dtype=jnp.int32) % N_NODES
    src = jnp.concatenate([edge_index[0].astype(jnp.int32), pad_idx])
    dst = jnp.concatenate([edge_index[1].astype(jnp.int32), pad_idx])
    ew = jnp.concatenate([edge_weight, jnp.zeros((n_pad,), jnp.float32)])
    src3 = src.reshape(NW, NCHUNK, 1, CK)
    dst3 = dst.reshape(NW, NCHUNK, 1, CK)
    w3 = ew.reshape(NW, NCHUNK, CK)
    w3r = jnp.broadcast_to(w3[..., None], (NW, NCHUNK, CK, 16))

    deg = _sc_deg(dst.reshape(NW, NB_D, 1, CK_D),
                  ew.reshape(NW, NB_D, 1, CK_D)).reshape(NC, DN_PAD)

    dinv, y1 = _t1(deg[0].reshape(DN_PAD, 1), deg[1].reshape(DN_PAD, 1), x)

    a = _sc_pass(y1, src3, dst3, w3r)
    a0, a1 = a[0], a[1]

    W23 = jnp.concatenate([W2, W3], axis=1)
    y2 = _t2(a0, a1, y1, dinv, W1, b1.reshape(1, H1), W23)

    cc = _sc_pass(y2, src3, dst3, w3r)
    c0, c1 = cc[0], cc[1]

    b23 = jnp.concatenate([b2, b3]).reshape(1, 2 * H2)
    z, mean, log_std = _t3(c0, c1, y2, dinv, b23, noise)
    return (z, mean, log_std)
